# baseline (speedup 1.0000x reference)
import functools

import jax
import jax.numpy as jnp
from jax import lax
from jax.experimental import pallas as pl
from jax.experimental.pallas import tpu as pltpu
from jax.experimental.pallas import tpu_sc as plsc

B = 4096
CTX = 20
E = 128
V = 100000

NC = 2
NS = 16
NW = NC * NS
BW = B // NW
ROWS_W = BW * CTX
IDX_MINOR = 128
N_DMA = ROWS_W // IDX_MINOR
GROUP_B = 32
N_GROUP = BW // GROUP_B
DMA_PER_GROUP = N_DMA // N_GROUP
GROUP_ROWS = GROUP_B * CTX


def _pool_body(idx_hbm, table_hbm, out_hbm, idx_v, rows_v, pooled_v, sem):
    wid = lax.axis_index("s") * NC + lax.axis_index("c")
    base_b = wid * BW

    pltpu.sync_copy(idx_hbm.at[wid], idx_v)

    for g in range(N_GROUP):
        copies = []
        for j in range(DMA_PER_GROUP):
            cp = pltpu.make_async_copy(
                table_hbm.at[idx_v.at[g * DMA_PER_GROUP + j]],
                rows_v.at[pl.ds(j * IDX_MINOR, IDX_MINOR)],
                sem,
            )
            cp.start()
            copies.append(cp)
        for cp in copies:
            cp.wait()

        def one_batch(i, carry):
            accs = None
            for l in range(CTX):
                vals = [rows_v[i * CTX + l, pl.ds(c * 16, 16)]
                        for c in range(8)]
                accs = vals if accs is None else [a + v
                                                 for a, v in zip(accs, vals)]
            for c in range(8):
                pooled_v[i, pl.ds(c * 16, 16)] = accs[c] * jnp.float32(1.0 / CTX)
            return carry

        lax.fori_loop(0, GROUP_B, one_batch, 0)

        pltpu.sync_copy(pooled_v, out_hbm.at[pl.ds(base_b + g * GROUP_B, GROUP_B)])


def _sc_pool(idx3, table):
    mesh = plsc.VectorSubcoreMesh(core_axis_name="c", subcore_axis_name="s")
    kern = pl.kernel(
        _pool_body,
        out_type=jax.ShapeDtypeStruct((B, E), jnp.float32),
        mesh=mesh,
        scratch_types=[
            pltpu.VMEM((N_DMA, IDX_MINOR), jnp.int32),
            pltpu.VMEM((GROUP_ROWS, E), jnp.float32),
            pltpu.VMEM((GROUP_B, E), jnp.float32),
            pltpu.SemaphoreType.DMA,
        ],
    )
    return kern(idx3, table)


BV = 12544
BB = 256


def _mm_body(p_ref, w_ref, b_ref, o_ref):
    p = p_ref[...].astype(jnp.bfloat16)
    w = w_ref[...].astype(jnp.bfloat16)
    acc = lax.dot_general(p, w, (((1,), (1,)), ((), ())),
                          preferred_element_type=jnp.float32)
    o_ref[...] = acc + b_ref[...]


def _tc_matmul(pooled, W, b):
    nv = pl.cdiv(V, BV)
    nb = B // BB
    return pl.pallas_call(
        _mm_body,
        grid=(nv, nb),
        in_specs=[
            pl.BlockSpec((BB, E), lambda j, i: (i, 0)),
            pl.BlockSpec((BV, E), lambda j, i: (j, 0)),
            pl.BlockSpec((1, BV), lambda j, i: (0, j)),
        ],
        out_specs=pl.BlockSpec((BB, BV), lambda j, i: (i, j)),
        out_shape=jax.ShapeDtypeStruct((B, V), jnp.float32),
    )(pooled, W, b.reshape(1, V))


def kernel(input_tensor, embedding_table, W, b):
    idx3 = input_tensor.reshape(NW, N_DMA, IDX_MINOR).astype(jnp.int32)
    pooled = _sc_pool(idx3, embedding_table)
    return _tc_matmul(pooled, W, b)


# speedup vs baseline: 3.4781x; 3.4781x over previous
import functools

import jax
import jax.numpy as jnp
from jax import lax
from jax.experimental import pallas as pl
from jax.experimental.pallas import tpu as pltpu
from jax.experimental.pallas import tpu_sc as plsc

B = 4096
CTX = 20
E = 128
V = 100000

NC = 2
NS = 16
NW = NC * NS
BW = B // NW
ROWS_W = BW * CTX
IDX_MINOR = 128
N_DMA = ROWS_W // IDX_MINOR
GROUP_B = 32
N_GROUP = BW // GROUP_B
DMA_PER_GROUP = N_DMA // N_GROUP
GROUP_ROWS = GROUP_B * CTX


def _pool_body(idx_hbm, table_hbm, out_hbm, idx_v, rows_v, pooled_v, sem):
    wid = lax.axis_index("s") * NC + lax.axis_index("c")
    base_b = wid * BW

    pltpu.sync_copy(idx_hbm.at[wid], idx_v)

    for g in range(N_GROUP):
        copies = []
        for j in range(DMA_PER_GROUP):
            cp = pltpu.make_async_copy(
                table_hbm.at[idx_v.at[g * DMA_PER_GROUP + j]],
                rows_v.at[pl.ds(j * IDX_MINOR, IDX_MINOR)],
                sem,
            )
            cp.start()
            copies.append(cp)
        for cp in copies:
            cp.wait()

        def one_batch(i, carry):
            accs = None
            for l in range(CTX):
                vals = [rows_v[i * CTX + l, pl.ds(c * 16, 16)]
                        for c in range(8)]
                accs = vals if accs is None else [a + v
                                                 for a, v in zip(accs, vals)]
            for c in range(8):
                pooled_v[i, pl.ds(c * 16, 16)] = accs[c] * jnp.float32(1.0 / CTX)
            return carry

        lax.fori_loop(0, GROUP_B, one_batch, 0)

        pltpu.sync_copy(pooled_v, out_hbm.at[pl.ds(base_b + g * GROUP_B, GROUP_B)])


def _sc_pool(idx3, table):
    mesh = plsc.VectorSubcoreMesh(core_axis_name="c", subcore_axis_name="s")
    kern = pl.kernel(
        _pool_body,
        out_type=jax.ShapeDtypeStruct((B, E), jnp.float32),
        mesh=mesh,
        scratch_types=[
            pltpu.VMEM((N_DMA, IDX_MINOR), jnp.int32),
            pltpu.VMEM((GROUP_ROWS, E), jnp.float32),
            pltpu.VMEM((GROUP_B, E), jnp.float32),
            pltpu.SemaphoreType.DMA,
        ],
    )
    return kern(idx3, table)


BVT = 512


def _mm_body(p_ref, w_ref, b_ref, o_ref):
    w = w_ref[...].astype(jnp.bfloat16)
    acc = lax.dot_general(w, p_ref[...], (((1,), (1,)), ((), ())),
                          preferred_element_type=jnp.float32)
    o_ref[...] = acc + b_ref[...][:, None]


def _tc_matmul(pooled, W, b):
    nv = pl.cdiv(V, BVT)
    out_t = pl.pallas_call(
        _mm_body,
        grid=(nv,),
        in_specs=[
            pl.BlockSpec((B, E), lambda j: (0, 0)),
            pl.BlockSpec((BVT, E), lambda j: (j, 0)),
            pl.BlockSpec((BVT,), lambda j: (j,)),
        ],
        out_specs=pl.BlockSpec((BVT, B), lambda j: (j, 0)),
        out_shape=jax.ShapeDtypeStruct((V, B), jnp.float32),
    )(pooled.astype(jnp.bfloat16), W, b)
    return out_t.T


def kernel(input_tensor, embedding_table, W, b):
    idx3 = input_tensor.reshape(NW, N_DMA, IDX_MINOR).astype(jnp.int32)
    pooled = _sc_pool(idx3, embedding_table)
    return _tc_matmul(pooled, W, b)


# speedup vs baseline: 3.5345x; 1.0162x over previous
import functools

import jax
import jax.numpy as jnp
from jax import lax
from jax.experimental import pallas as pl
from jax.experimental.pallas import tpu as pltpu
from jax.experimental.pallas import tpu_sc as plsc

B = 4096
CTX = 20
E = 128
V = 100000

NC = 2
NS = 16
NW = NC * NS
BW = B // NW
ROWS_W = BW * CTX
IDX_MINOR = 128
N_DMA = ROWS_W // IDX_MINOR
GROUP_B = 32
N_GROUP = BW // GROUP_B
DMA_PER_GROUP = N_DMA // N_GROUP
GROUP_ROWS = GROUP_B * CTX


def _pool_body(idx_hbm, table_hbm, out_hbm, idx_v, rows_v, pooled_v, sem):
    wid = lax.axis_index("s") * NC + lax.axis_index("c")
    base_b = wid * BW

    pltpu.sync_copy(idx_hbm.at[wid], idx_v)

    for g in range(N_GROUP):
        copies = []
        for j in range(DMA_PER_GROUP):
            cp = pltpu.make_async_copy(
                table_hbm.at[idx_v.at[g * DMA_PER_GROUP + j]],
                rows_v.at[pl.ds(j * IDX_MINOR, IDX_MINOR)],
                sem,
            )
            cp.start()
            copies.append(cp)
        for cp in copies:
            cp.wait()

        def one_batch(i, carry):
            accs = None
            for l in range(CTX):
                vals = [rows_v[i * CTX + l, pl.ds(c * 16, 16)]
                        for c in range(8)]
                accs = vals if accs is None else [a + v
                                                 for a, v in zip(accs, vals)]
            for c in range(8):
                pooled_v[i, pl.ds(c * 16, 16)] = accs[c] * jnp.float32(1.0 / CTX)
            return carry

        lax.fori_loop(0, GROUP_B, one_batch, 0)

        pltpu.sync_copy(pooled_v, out_hbm.at[pl.ds(base_b + g * GROUP_B, GROUP_B)])


def _sc_pool(idx3, table):
    mesh = plsc.VectorSubcoreMesh(core_axis_name="c", subcore_axis_name="s")
    kern = pl.kernel(
        _pool_body,
        out_type=jax.ShapeDtypeStruct((B, E), jnp.float32),
        mesh=mesh,
        scratch_types=[
            pltpu.VMEM((N_DMA, IDX_MINOR), jnp.int32),
            pltpu.VMEM((GROUP_ROWS, E), jnp.float32),
            pltpu.VMEM((GROUP_B, E), jnp.float32),
            pltpu.SemaphoreType.DMA,
        ],
    )
    return kern(idx3, table)


BVT = 1024


def _mm_body(p_ref, w_ref, b_ref, o_ref):
    w = w_ref[...].astype(jnp.bfloat16)
    acc = lax.dot_general(w, p_ref[...], (((1,), (1,)), ((), ())),
                          preferred_element_type=jnp.float32)
    o_ref[...] = acc + b_ref[...][:, None]


def _tc_matmul(pooled, W, b):
    nv = pl.cdiv(V, BVT)
    out_t = pl.pallas_call(
        _mm_body,
        grid=(nv,),
        in_specs=[
            pl.BlockSpec((B, E), lambda j: (0, 0)),
            pl.BlockSpec((BVT, E), lambda j: (j, 0)),
            pl.BlockSpec((BVT,), lambda j: (j,)),
        ],
        out_specs=pl.BlockSpec((BVT, B), lambda j: (j, 0)),
        out_shape=jax.ShapeDtypeStruct((V, B), jnp.float32),
    )(pooled.astype(jnp.bfloat16), W, b)
    return out_t.T


def kernel(input_tensor, embedding_table, W, b):
    idx3 = input_tensor.reshape(NW, N_DMA, IDX_MINOR).astype(jnp.int32)
    pooled = _sc_pool(idx3, embedding_table)
    return _tc_matmul(pooled, W, b)


# speedup vs baseline: 3.5829x; 1.0137x over previous
import jax
import jax.numpy as jnp
from jax import lax
from jax.experimental import pallas as pl
from jax.experimental.pallas import tpu as pltpu
from jax.experimental.pallas import tpu_sc as plsc

B = 4096
CTX = 20
E = 128
V = 100000

NC = 2
NS = 16
NW = NC * NS
BW = B // NW
ROWS_W = BW * CTX
GROUP_B = 16
N_GROUP = BW // GROUP_B
GROUP_ROWS = GROUP_B * CTX
DMA_PER_GROUP = 4
IDX_MINOR = GROUP_ROWS // DMA_PER_GROUP
N_DMA = ROWS_W // IDX_MINOR


def _pool_body(idx_hbm, table_hbm, out_hbm, idx_v, rows_a, rows_b, pooled_v,
               sem_a, sem_b):
    wid = lax.axis_index("s") * NC + lax.axis_index("c")
    base_b = wid * BW

    pltpu.sync_copy(idx_hbm.at[wid], idx_v)

    bufs = (rows_a, rows_b)
    sems = (sem_a, sem_b)

    def fire(g, slot):
        for j in range(DMA_PER_GROUP):
            pltpu.make_async_copy(
                table_hbm.at[idx_v.at[g * DMA_PER_GROUP + j]],
                bufs[slot].at[pl.ds(j * IDX_MINOR, IDX_MINOR)],
                sems[slot],
            ).start()

    def drain(slot):
        for j in range(DMA_PER_GROUP):
            pltpu.make_async_copy(
                table_hbm.at[idx_v.at[j]],
                bufs[slot].at[pl.ds(j * IDX_MINOR, IDX_MINOR)],
                sems[slot],
            ).wait()

    fire(0, 0)
    for g in range(N_GROUP):
        slot = g % 2
        if g + 1 < N_GROUP:
            fire(g + 1, 1 - slot)
        drain(slot)
        rows_v = bufs[slot]

        def one_batch(i, carry):
            accs = None
            for l in range(CTX):
                vals = [rows_v[i * CTX + l, pl.ds(c * 16, 16)]
                        for c in range(8)]
                accs = vals if accs is None else [a + v
                                                 for a, v in zip(accs, vals)]
            for c in range(8):
                pooled_v[i, pl.ds(c * 16, 16)] = accs[c] * jnp.float32(1.0 / CTX)
            return carry

        lax.fori_loop(0, GROUP_B, one_batch, 0)

        pltpu.sync_copy(pooled_v,
                        out_hbm.at[pl.ds(base_b + g * GROUP_B, GROUP_B)])


def _sc_pool(idx3, table):
    mesh = plsc.VectorSubcoreMesh(core_axis_name="c", subcore_axis_name="s")
    kern = pl.kernel(
        _pool_body,
        out_type=jax.ShapeDtypeStruct((B, E), jnp.float32),
        mesh=mesh,
        scratch_types=[
            pltpu.VMEM((N_DMA, IDX_MINOR), jnp.int32),
            pltpu.VMEM((GROUP_ROWS, E), jnp.float32),
            pltpu.VMEM((GROUP_ROWS, E), jnp.float32),
            pltpu.VMEM((GROUP_B, E), jnp.float32),
            pltpu.SemaphoreType.DMA,
            pltpu.SemaphoreType.DMA,
        ],
    )
    return kern(idx3, table)


BVT = 1536


def _mm_body(p_ref, w_ref, b_ref, o_ref):
    w = w_ref[...].astype(jnp.bfloat16)
    acc = lax.dot_general(w, p_ref[...], (((1,), (1,)), ((), ())),
                          preferred_element_type=jnp.float32)
    o_ref[...] = acc + b_ref[0, 0, :][:, None]


def _tc_matmul(pooled, W, b):
    nv = pl.cdiv(V, BVT)
    b2 = jnp.pad(b, (0, nv * BVT - V)).reshape(nv, 1, BVT)
    out_t = pl.pallas_call(
        _mm_body,
        grid=(nv,),
        in_specs=[
            pl.BlockSpec((B, E), lambda j: (0, 0)),
            pl.BlockSpec((BVT, E), lambda j: (j, 0)),
            pl.BlockSpec((1, 1, BVT), lambda j: (j, 0, 0)),
        ],
        out_specs=pl.BlockSpec((BVT, B), lambda j: (j, 0)),
        out_shape=jax.ShapeDtypeStruct((V, B), jnp.float32),
    )(pooled.astype(jnp.bfloat16), W, b2)
    return out_t.T


def kernel(input_tensor, embedding_table, W, b):
    idx3 = input_tensor.reshape(NW, N_DMA, IDX_MINOR).astype(jnp.int32)
    pooled = _sc_pool(idx3, embedding_table)
    return _tc_matmul(pooled, W, b)
